# Initial kernel scaffold; baseline (speedup 1.0000x reference)
import jax
import jax.numpy as jnp
from jax.experimental import pallas as pl


def kernel(target_word_ids, context_word_ids, in_embedding, out_embedding):
    raise NotImplementedError("write your pallas kernel here")


# speedup vs baseline: 1.3812x; 1.3812x over previous
import functools

import jax
import jax.numpy as jnp
from jax import lax
from jax.experimental import pallas as pl
from jax.experimental.pallas import tpu as pltpu
from jax.experimental.pallas import tpu_sc as plsc

VOCAB = 1000000
DIM = 64
BATCH = 16384
CTX = 50

NUM_CORES = 2
NUM_SUBCORES = 16
NUM_WORKERS = NUM_CORES * NUM_SUBCORES
ROWS_PER_WORKER = BATCH // NUM_WORKERS
CHUNK_ROWS = 16
CHUNK_PAIRS = CHUNK_ROWS * CTX
NUM_CHUNKS = ROWS_PER_WORKER // CHUNK_ROWS
GATHER_SLICE = 128
LANES = 16


def _sc_body(tgt_ids_hbm, ctx_ids_hbm, in_emb_hbm, out_emb_hbm, out_hbm,
             tgt_idx_v, ctx_idx_v, tgt_rows_v, ctx_rows_v, out_v, sem, gsem):
    wid = lax.axis_index("s") * NUM_CORES + lax.axis_index("c")
    row0 = wid * ROWS_PER_WORKER

    lane_iota = lax.iota(jnp.int32, LANES)
    last_lane = lane_iota == (LANES - 1)

    def chunk_body(chunk, _):
        rowbase = row0 + chunk * CHUNK_ROWS
        pairbase = rowbase * CTX

        pltpu.sync_copy(tgt_ids_hbm.at[pl.ds(rowbase, CHUNK_ROWS)], tgt_idx_v)
        pltpu.sync_copy(ctx_ids_hbm.at[pl.ds(pairbase, CHUNK_PAIRS)], ctx_idx_v)

        pltpu.async_copy(in_emb_hbm.at[tgt_idx_v], tgt_rows_v, sem)
        copies = []
        off = 0
        while off < CHUNK_PAIRS:
            size = min(GATHER_SLICE, CHUNK_PAIRS - off)
            copies.append(pltpu.async_copy(
                out_emb_hbm.at[ctx_idx_v.at[pl.ds(off, size)]],
                ctx_rows_v.at[pl.ds(off, size)], gsem))
            off += size
        pltpu.make_async_copy(in_emb_hbm.at[tgt_idx_v], tgt_rows_v, sem).wait()
        for c in copies:
            c.wait()

        def row_body(r, _):
            t0 = tgt_rows_v[r, pl.ds(0, LANES)]
            t1 = tgt_rows_v[r, pl.ds(LANES, LANES)]
            t2 = tgt_rows_v[r, pl.ds(2 * LANES, LANES)]
            t3 = tgt_rows_v[r, pl.ds(3 * LANES, LANES)]
            p0 = r * CTX
            for l in range(CTX):
                p = p0 + l
                c0 = ctx_rows_v[p, pl.ds(0, LANES)]
                c1 = ctx_rows_v[p, pl.ds(LANES, LANES)]
                c2 = ctx_rows_v[p, pl.ds(2 * LANES, LANES)]
                c3 = ctx_rows_v[p, pl.ds(3 * LANES, LANES)]
                prod = (c0 * t0 + c1 * t1) + (c2 * t2 + c3 * t3)
                cum = plsc.cumsum(prod)
                plsc.store_scatter(out_v, [jnp.full((LANES,), p, jnp.int32)],
                                   cum, mask=last_lane)
            return ()

        lax.fori_loop(0, CHUNK_ROWS, row_body, (), unroll=False)

        def sig_body(i, _):
            x = out_v[pl.ds(i * LANES, LANES)]
            out_v[pl.ds(i * LANES, LANES)] = 1.0 / (1.0 + jnp.exp(-x))
            return ()

        lax.fori_loop(0, CHUNK_PAIRS // LANES, sig_body, (), unroll=False)

        pltpu.sync_copy(out_v, out_hbm.at[pl.ds(pairbase, CHUNK_PAIRS)])
        return ()

    lax.fori_loop(0, NUM_CHUNKS, chunk_body, (), unroll=False)


@jax.jit
def _run(tgt_ids, ctx_ids, in_emb, out_emb):
    mesh = plsc.VectorSubcoreMesh(core_axis_name="c", subcore_axis_name="s")
    kfn = pl.kernel(
        _sc_body,
        mesh=mesh,
        out_type=jax.ShapeDtypeStruct((BATCH * CTX,), jnp.float32),
        scratch_types=[
            pltpu.VMEM((CHUNK_ROWS,), jnp.int32),
            pltpu.VMEM((CHUNK_PAIRS,), jnp.int32),
            pltpu.VMEM((CHUNK_ROWS, DIM), jnp.float32),
            pltpu.VMEM((CHUNK_PAIRS, DIM), jnp.float32),
            pltpu.VMEM((CHUNK_PAIRS,), jnp.float32),
            pltpu.SemaphoreType.DMA,
            pltpu.SemaphoreType.DMA,
        ],
        compiler_params=pltpu.CompilerParams(
            needs_layout_passes=False, use_tc_tiling_on_sc=False),
    )
    return kfn(tgt_ids, ctx_ids, in_emb, out_emb)


def kernel(target_word_ids, context_word_ids, in_embedding, out_embedding):
    tgt = target_word_ids.reshape(BATCH).astype(jnp.int32)
    ctx = context_word_ids.reshape(BATCH * CTX).astype(jnp.int32)
    flat = _run(tgt, ctx, in_embedding, out_embedding)
    return flat.reshape(BATCH, CTX)


# speedup vs baseline: 1.6367x; 1.1850x over previous
import jax
import jax.numpy as jnp
from jax import lax
from jax.experimental import pallas as pl
from jax.experimental.pallas import tpu as pltpu
from jax.experimental.pallas import tpu_sc as plsc

VOCAB = 1000000
DIM = 64
BATCH = 16384
CTX = 50

NUM_CORES = 2
NUM_SUBCORES = 16
NUM_WORKERS = NUM_CORES * NUM_SUBCORES
ROWS_PER_WORKER = BATCH // NUM_WORKERS
CHUNK_ROWS = 8
CHUNK_PAIRS = CHUNK_ROWS * CTX
NUM_CHUNKS = ROWS_PER_WORKER // CHUNK_ROWS
LANES = 16
NGROUPS = CHUNK_PAIRS // LANES
CTX_SLICES = [(0, 128), (128, 128), (256, 128), (384, 16)]


def _tree_reduce16(prods, iota):
    vecs = prods
    k = 1
    while len(vecs) > 1:
        perm = iota ^ k
        mask = (iota & k) == 0
        nxt = []
        for i in range(0, len(vecs), 2):
            a, b = vecs[i], vecs[i + 1]
            a_s = jnp.take_along_axis(a, perm, axis=0)
            b_s = jnp.take_along_axis(b, perm, axis=0)
            nxt.append(jnp.where(mask, a, b_s) + jnp.where(mask, a_s, b))
        vecs = nxt
        k *= 2
    return vecs[0]


def _sc_body(tgt_ids_hbm, ctx_ids_hbm, in_emb_hbm, out_emb_hbm, out_hbm,
             tgt_idx_v, ctx_idx_v, tgt_rows_v, ctx_rows_v, out_v,
             semt0, semt1, semc0, semc1):
    wid = lax.axis_index("s") * NUM_CORES + lax.axis_index("c")
    row0 = wid * ROWS_PER_WORKER
    sem_t = (semt0, semt1)
    sem_c = (semc0, semc1)

    iota = lax.iota(jnp.int32, LANES)

    def issue(c, nb):
        rowbase = row0 + c * CHUNK_ROWS
        pairbase = rowbase * CTX
        pltpu.sync_copy(tgt_ids_hbm.at[pl.ds(rowbase, CHUNK_ROWS)],
                        tgt_idx_v.at[nb])
        pltpu.sync_copy(ctx_ids_hbm.at[pl.ds(pairbase, CHUNK_PAIRS)],
                        ctx_idx_v.at[nb])
        pltpu.async_copy(in_emb_hbm.at[tgt_idx_v.at[nb]], tgt_rows_v.at[nb],
                         sem_t[nb])
        for off, size in CTX_SLICES:
            pltpu.async_copy(
                out_emb_hbm.at[ctx_idx_v.at[nb, pl.ds(off, size)]],
                ctx_rows_v.at[nb, pl.ds(off, size)], sem_c[nb])

    def wait_bufs(b):
        pltpu.make_async_copy(in_emb_hbm.at[tgt_idx_v.at[b]],
                              tgt_rows_v.at[b], sem_t[b]).wait()
        pltpu.make_async_copy(out_emb_hbm.at[ctx_idx_v.at[b]],
                              ctx_rows_v.at[b], sem_c[b]).wait()

    def compute(c, b):
        ctx_r = ctx_rows_v.at[b]
        tgt_r = tgt_rows_v.at[b]
        tcache = {}
        for g in range(NGROUPS):
            prods = []
            for i in range(LANES):
                p = g * LANES + i
                r = p // CTX
                if r not in tcache:
                    tcache[r] = tuple(
                        tgt_r[r, pl.ds(j * LANES, LANES)] for j in range(4))
                t0, t1, t2, t3 = tcache[r]
                c0 = ctx_r[p, pl.ds(0, LANES)]
                c1 = ctx_r[p, pl.ds(LANES, LANES)]
                c2 = ctx_r[p, pl.ds(2 * LANES, LANES)]
                c3 = ctx_r[p, pl.ds(3 * LANES, LANES)]
                prods.append((c0 * t0 + c1 * t1) + (c2 * t2 + c3 * t3))
            res = _tree_reduce16(prods, iota)
            res = 1.0 / (1.0 + jnp.exp(-res))
            out_v[pl.ds(g * LANES, LANES)] = res
        pairbase = (row0 + c * CHUNK_ROWS) * CTX
        pltpu.sync_copy(out_v, out_hbm.at[pl.ds(pairbase, CHUNK_PAIRS)])

    issue(0, 0)

    def chunk2_body(g2, _):
        for b in range(2):
            c = g2 * 2 + b
            wait_bufs(b)

            @pl.when(c < NUM_CHUNKS - 1)
            def _():
                issue(c + 1, 1 - b)

            compute(c, b)
        return ()

    lax.fori_loop(0, NUM_CHUNKS // 2, chunk2_body, (), unroll=False)


@jax.jit
def _run(tgt_ids, ctx_ids, in_emb, out_emb):
    mesh = plsc.VectorSubcoreMesh(core_axis_name="c", subcore_axis_name="s")
    kfn = pl.kernel(
        _sc_body,
        mesh=mesh,
        out_type=jax.ShapeDtypeStruct((BATCH * CTX,), jnp.float32),
        scratch_types=[
            pltpu.VMEM((2, CHUNK_ROWS), jnp.int32),
            pltpu.VMEM((2, CHUNK_PAIRS), jnp.int32),
            pltpu.VMEM((2, CHUNK_ROWS, DIM), jnp.float32),
            pltpu.VMEM((2, CHUNK_PAIRS, DIM), jnp.float32),
            pltpu.VMEM((CHUNK_PAIRS,), jnp.float32),
            pltpu.SemaphoreType.DMA,
            pltpu.SemaphoreType.DMA,
            pltpu.SemaphoreType.DMA,
            pltpu.SemaphoreType.DMA,
        ],
        compiler_params=pltpu.CompilerParams(
            needs_layout_passes=False, use_tc_tiling_on_sc=False),
    )
    return kfn(tgt_ids, ctx_ids, in_emb, out_emb)


def kernel(target_word_ids, context_word_ids, in_embedding, out_embedding):
    tgt = target_word_ids.reshape(BATCH).astype(jnp.int32)
    ctx = context_word_ids.reshape(BATCH * CTX).astype(jnp.int32)
    flat = _run(tgt, ctx, in_embedding, out_embedding)
    return flat.reshape(BATCH, CTX)


# speedup vs baseline: 3.2911x; 2.0109x over previous
import jax
import jax.numpy as jnp
from jax import lax
from jax.experimental import pallas as pl
from jax.experimental.pallas import tpu as pltpu
from jax.experimental.pallas import tpu_sc as plsc

VOCAB = 1000000
DIM = 64
BATCH = 16384
CTX = 50

NUM_CORES = 2
NUM_SUBCORES = 16
NUM_WORKERS = NUM_CORES * NUM_SUBCORES
ROWS_PER_WORKER = BATCH // NUM_WORKERS
CHUNK_ROWS = 16
CHUNK_PAIRS = CHUNK_ROWS * CTX
NUM_CHUNKS = ROWS_PER_WORKER // CHUNK_ROWS
LANES = 16
NGROUPS = CHUNK_PAIRS // LANES
CTX_SLICES = [(i * 128, 128) for i in range(6)] + [(768, 32)]


def _tree_reduce16(prods, iota):
    vecs = prods
    k = 1
    while len(vecs) > 1:
        perm = iota ^ k
        mask = (iota & k) == 0
        nxt = []
        for i in range(0, len(vecs), 2):
            a, b = vecs[i], vecs[i + 1]
            a_s = jnp.take_along_axis(a, perm, axis=0)
            b_s = jnp.take_along_axis(b, perm, axis=0)
            nxt.append(jnp.where(mask, a, b_s) + jnp.where(mask, a_s, b))
        vecs = nxt
        k *= 2
    return vecs[0]


def _sc_body(tgt_ids_hbm, ctx_ids_hbm, in_emb_hbm, out_emb_hbm, out_hbm,
             tgt_idx_v, ctx_idx_v, tgt_rows_v, ctx_rows_v, out_v,
             semt0, semt1, semc0, semc1, semi0, semi1, semo0, semo1):
    wid = lax.axis_index("s") * NUM_CORES + lax.axis_index("c")
    row0 = wid * ROWS_PER_WORKER
    sem_t = (semt0, semt1)
    sem_c = (semc0, semc1)
    sem_i = (semi0, semi1)
    sem_o = (semo0, semo1)

    iota = lax.iota(jnp.int32, LANES)

    def issue_ids(c, nb):
        rowbase = row0 + c * CHUNK_ROWS
        pairbase = rowbase * CTX
        pltpu.async_copy(tgt_ids_hbm.at[pl.ds(rowbase, CHUNK_ROWS)],
                         tgt_idx_v.at[nb], sem_i[nb])
        pltpu.async_copy(ctx_ids_hbm.at[pl.ds(pairbase, CHUNK_PAIRS)],
                         ctx_idx_v.at[nb], sem_i[nb])

    def wait_ids(c, nb):
        rowbase = row0 + c * CHUNK_ROWS
        pairbase = rowbase * CTX
        pltpu.make_async_copy(tgt_ids_hbm.at[pl.ds(rowbase, CHUNK_ROWS)],
                              tgt_idx_v.at[nb], sem_i[nb]).wait()
        pltpu.make_async_copy(ctx_ids_hbm.at[pl.ds(pairbase, CHUNK_PAIRS)],
                              ctx_idx_v.at[nb], sem_i[nb]).wait()

    def issue_gathers(nb):
        pltpu.async_copy(in_emb_hbm.at[tgt_idx_v.at[nb]], tgt_rows_v.at[nb],
                         sem_t[nb])
        for off, size in CTX_SLICES:
            pltpu.async_copy(
                out_emb_hbm.at[ctx_idx_v.at[nb, pl.ds(off, size)]],
                ctx_rows_v.at[nb, pl.ds(off, size)], sem_c[nb])

    def wait_rows(b):
        pltpu.make_async_copy(in_emb_hbm.at[tgt_idx_v.at[b]],
                              tgt_rows_v.at[b], sem_t[b]).wait()
        pltpu.make_async_copy(out_emb_hbm.at[ctx_idx_v.at[b]],
                              ctx_rows_v.at[b], sem_c[b]).wait()

    def out_slice(c):
        return out_hbm.at[pl.ds((row0 + c * CHUNK_ROWS) * CTX, CHUNK_PAIRS)]

    def compute(c, b):
        ctx_r = ctx_rows_v.at[b]
        tgt_r = tgt_rows_v.at[b]
        tcache = {}
        for g in range(NGROUPS):
            prods = []
            for i in range(LANES):
                p = g * LANES + i
                r = p // CTX
                if r not in tcache:
                    tcache[r] = tuple(
                        tgt_r[r, pl.ds(j * LANES, LANES)] for j in range(4))
                t0, t1, t2, t3 = tcache[r]
                c0 = ctx_r[p, pl.ds(0, LANES)]
                c1 = ctx_r[p, pl.ds(LANES, LANES)]
                c2 = ctx_r[p, pl.ds(2 * LANES, LANES)]
                c3 = ctx_r[p, pl.ds(3 * LANES, LANES)]
                prods.append((c0 * t0 + c1 * t1) + (c2 * t2 + c3 * t3))
            res = _tree_reduce16(prods, iota)
            res = 1.0 / (1.0 + jnp.exp(-res))
            out_v[b, pl.ds(g * LANES, LANES)] = res
        pltpu.async_copy(out_v.at[b], out_slice(c), sem_o[b])

    issue_ids(0, 0)
    issue_ids(1, 1)
    wait_ids(0, 0)
    issue_gathers(0)

    def chunk2_body(g2, _):
        for b in range(2):
            c = g2 * 2 + b
            wait_rows(b)

            @pl.when(c < NUM_CHUNKS - 1)
            def _():
                wait_ids(c + 1, 1 - b)
                issue_gathers(1 - b)

            @pl.when(c < NUM_CHUNKS - 2)
            def _():
                issue_ids(c + 2, b)

            @pl.when(c >= 2)
            def _():
                pltpu.make_async_copy(out_v.at[b], out_slice(c - 2),
                                      sem_o[b]).wait()

            compute(c, b)
        return ()

    lax.fori_loop(0, NUM_CHUNKS // 2, chunk2_body, (), unroll=False)

    pltpu.make_async_copy(out_v.at[0], out_slice(NUM_CHUNKS - 2),
                          sem_o[0]).wait()
    pltpu.make_async_copy(out_v.at[1], out_slice(NUM_CHUNKS - 1),
                          sem_o[1]).wait()


@jax.jit
def _run(tgt_ids, ctx_ids, in_emb, out_emb):
    mesh = plsc.VectorSubcoreMesh(core_axis_name="c", subcore_axis_name="s")
    kfn = pl.kernel(
        _sc_body,
        mesh=mesh,
        out_type=jax.ShapeDtypeStruct((BATCH * CTX,), jnp.float32),
        scratch_types=[
            pltpu.VMEM((2, CHUNK_ROWS), jnp.int32),
            pltpu.VMEM((2, CHUNK_PAIRS), jnp.int32),
            pltpu.VMEM((2, CHUNK_ROWS, DIM), jnp.float32),
            pltpu.VMEM((2, CHUNK_PAIRS, DIM), jnp.float32),
            pltpu.VMEM((2, CHUNK_PAIRS), jnp.float32),
            pltpu.SemaphoreType.DMA,
            pltpu.SemaphoreType.DMA,
            pltpu.SemaphoreType.DMA,
            pltpu.SemaphoreType.DMA,
            pltpu.SemaphoreType.DMA,
            pltpu.SemaphoreType.DMA,
            pltpu.SemaphoreType.DMA,
            pltpu.SemaphoreType.DMA,
        ],
        compiler_params=pltpu.CompilerParams(
            needs_layout_passes=False, use_tc_tiling_on_sc=False),
    )
    return kfn(tgt_ids, ctx_ids, in_emb, out_emb)


TBLK = 16384
NBLK = (VOCAB + TBLK - 1) // TBLK
HBLK = (NBLK + 1) // 2
HALF = HBLK * TBLK


def _tc_transpose_body(lo_ref, hi_ref, dst_ref):
    dst_ref[:, 0:DIM] = lo_ref[...].T
    dst_ref[:, DIM:2 * DIM] = hi_ref[...].T


def _linearize(table):
    t = table.T
    packed = pl.pallas_call(
        _tc_transpose_body,
        grid=(HBLK,),
        in_specs=[pl.BlockSpec((DIM, TBLK), lambda i: (0, i)),
                  pl.BlockSpec((DIM, TBLK), lambda i: (0, i + HBLK))],
        out_specs=pl.BlockSpec((TBLK, 2 * DIM), lambda i: (i, 0)),
        out_shape=jax.ShapeDtypeStruct((HALF, 2 * DIM), jnp.float32),
    )(t, t)
    return packed.reshape(2 * HALF, DIM)


def kernel(target_word_ids, context_word_ids, in_embedding, out_embedding):
    tgt = target_word_ids.reshape(BATCH).astype(jnp.int32)
    ctx = context_word_ids.reshape(BATCH * CTX).astype(jnp.int32)
    tgt = jnp.where(tgt < HALF, 2 * tgt, 2 * (tgt - HALF) + 1)
    ctx = jnp.where(ctx < HALF, 2 * ctx, 2 * (ctx - HALF) + 1)
    flat = _run(tgt, ctx, _linearize(in_embedding), _linearize(out_embedding))
    return flat.reshape(BATCH, CTX)


# speedup vs baseline: 3.4349x; 1.0437x over previous
import jax
import jax.numpy as jnp
from jax import lax
from jax.experimental import pallas as pl
from jax.experimental.pallas import tpu as pltpu
from jax.experimental.pallas import tpu_sc as plsc

VOCAB = 1000000
DIM = 64
BATCH = 16384
CTX = 50

NUM_CORES = 2
NUM_SUBCORES = 16
NUM_WORKERS = NUM_CORES * NUM_SUBCORES
ROWS_PER_WORKER = BATCH // NUM_WORKERS
CHUNK_ROWS = 8
CHUNK_PAIRS = CHUNK_ROWS * CTX
NUM_CHUNKS = ROWS_PER_WORKER // CHUNK_ROWS
LANES = 16
NGROUPS = CHUNK_PAIRS // LANES
CTX_SLICES = [(0, 128), (128, 128), (256, 128), (384, 16)]


def _tree_reduce16(prods, iota):
    vecs = prods
    k = 1
    while len(vecs) > 1:
        perm = iota ^ k
        mask = (iota & k) == 0
        nxt = []
        for i in range(0, len(vecs), 2):
            a, b = vecs[i], vecs[i + 1]
            a_s = jnp.take_along_axis(a, perm, axis=0)
            b_s = jnp.take_along_axis(b, perm, axis=0)
            nxt.append(jnp.where(mask, a, b_s) + jnp.where(mask, a_s, b))
        vecs = nxt
        k *= 2
    return vecs[0]


def _sc_body(tgt_ids_hbm, ctx_ids_hbm, in_emb_hbm, out_emb_hbm, out_hbm,
             tgt_idx_v, ctx_idx_v, tgt_rows_v, ctx_rows_v, out_v,
             semt0, semt1, semc0, semc1, semi0, semi1, semo0, semo1):
    wid = lax.axis_index("s") * NUM_CORES + lax.axis_index("c")
    row0 = wid * ROWS_PER_WORKER
    sem_t = (semt0, semt1)
    sem_c = (semc0, semc1)
    sem_i = (semi0, semi1)
    sem_o = (semo0, semo1)

    iota = lax.iota(jnp.int32, LANES)

    def issue_ids(c, nb):
        rowbase = row0 + c * CHUNK_ROWS
        pairbase = rowbase * CTX
        pltpu.async_copy(tgt_ids_hbm.at[pl.ds(rowbase, CHUNK_ROWS)],
                         tgt_idx_v.at[nb], sem_i[nb])
        pltpu.async_copy(ctx_ids_hbm.at[pl.ds(pairbase, CHUNK_PAIRS)],
                         ctx_idx_v.at[nb], sem_i[nb])

    def wait_ids(c, nb):
        rowbase = row0 + c * CHUNK_ROWS
        pairbase = rowbase * CTX
        pltpu.make_async_copy(tgt_ids_hbm.at[pl.ds(rowbase, CHUNK_ROWS)],
                              tgt_idx_v.at[nb], sem_i[nb]).wait()
        pltpu.make_async_copy(ctx_ids_hbm.at[pl.ds(pairbase, CHUNK_PAIRS)],
                              ctx_idx_v.at[nb], sem_i[nb]).wait()

    def issue_gathers(nb):
        pltpu.async_copy(in_emb_hbm.at[tgt_idx_v.at[nb]], tgt_rows_v.at[nb],
                         sem_t[nb])
        for off, size in CTX_SLICES:
            pltpu.async_copy(
                out_emb_hbm.at[ctx_idx_v.at[nb, pl.ds(off, size)]],
                ctx_rows_v.at[nb, pl.ds(off, size)], sem_c[nb])

    def wait_rows(b):
        pltpu.make_async_copy(in_emb_hbm.at[tgt_idx_v.at[b]],
                              tgt_rows_v.at[b], sem_t[b]).wait()
        pltpu.make_async_copy(out_emb_hbm.at[ctx_idx_v.at[b]],
                              ctx_rows_v.at[b], sem_c[b]).wait()

    def out_slice(c):
        return out_hbm.at[pl.ds((row0 + c * CHUNK_ROWS) * CTX, CHUNK_PAIRS)]

    def compute(c, b):
        ctx_r = ctx_rows_v.at[b]
        tgt_r = tgt_rows_v.at[b]
        tcache = {}
        for g in range(NGROUPS):
            prods = []
            for i in range(LANES):
                p = g * LANES + i
                r = p // CTX
                if r not in tcache:
                    tcache[r] = tuple(
                        tgt_r[r, pl.ds(j * LANES, LANES)] for j in range(4))
                t0, t1, t2, t3 = tcache[r]
                c0 = ctx_r[p, pl.ds(0, LANES)]
                c1 = ctx_r[p, pl.ds(LANES, LANES)]
                c2 = ctx_r[p, pl.ds(2 * LANES, LANES)]
                c3 = ctx_r[p, pl.ds(3 * LANES, LANES)]
                prods.append((c0 * t0 + c1 * t1) + (c2 * t2 + c3 * t3))
            res = _tree_reduce16(prods, iota)
            res = 1.0 / (1.0 + jnp.exp(-res))
            out_v[b, pl.ds(g * LANES, LANES)] = res
        pltpu.async_copy(out_v.at[b], out_slice(c), sem_o[b])

    issue_ids(0, 0)
    issue_ids(1, 1)
    wait_ids(0, 0)
    issue_gathers(0)

    def chunk2_body(g2, _):
        for b in range(2):
            c = g2 * 2 + b
            wait_rows(b)

            @pl.when(c < NUM_CHUNKS - 1)
            def _():
                wait_ids(c + 1, 1 - b)
                issue_gathers(1 - b)

            @pl.when(c < NUM_CHUNKS - 2)
            def _():
                issue_ids(c + 2, b)

            @pl.when(c >= 2)
            def _():
                pltpu.make_async_copy(out_v.at[b], out_slice(c - 2),
                                      sem_o[b]).wait()

            compute(c, b)
        return ()

    lax.fori_loop(0, NUM_CHUNKS // 2, chunk2_body, (), unroll=False)

    pltpu.make_async_copy(out_v.at[0], out_slice(NUM_CHUNKS - 2),
                          sem_o[0]).wait()
    pltpu.make_async_copy(out_v.at[1], out_slice(NUM_CHUNKS - 1),
                          sem_o[1]).wait()


@jax.jit
def _run(tgt_ids, ctx_ids, in_emb, out_emb):
    mesh = plsc.VectorSubcoreMesh(core_axis_name="c", subcore_axis_name="s")
    kfn = pl.kernel(
        _sc_body,
        mesh=mesh,
        out_type=jax.ShapeDtypeStruct((BATCH * CTX,), jnp.float32),
        scratch_types=[
            pltpu.VMEM((2, CHUNK_ROWS), jnp.int32),
            pltpu.VMEM((2, CHUNK_PAIRS), jnp.int32),
            pltpu.VMEM((2, CHUNK_ROWS, DIM), jnp.float32),
            pltpu.VMEM((2, CHUNK_PAIRS, DIM), jnp.float32),
            pltpu.VMEM((2, CHUNK_PAIRS), jnp.float32),
            pltpu.SemaphoreType.DMA,
            pltpu.SemaphoreType.DMA,
            pltpu.SemaphoreType.DMA,
            pltpu.SemaphoreType.DMA,
            pltpu.SemaphoreType.DMA,
            pltpu.SemaphoreType.DMA,
            pltpu.SemaphoreType.DMA,
            pltpu.SemaphoreType.DMA,
        ],
        compiler_params=pltpu.CompilerParams(
            needs_layout_passes=False, use_tc_tiling_on_sc=False),
    )
    return kfn(tgt_ids, ctx_ids, in_emb, out_emb)


TBLK = 16384
NBLK = (VOCAB + TBLK - 1) // TBLK
HBLK = (NBLK + 1) // 2
HALF = HBLK * TBLK


def _tc_transpose_body(lo_ref, hi_ref, dst_ref):
    dst_ref[:, 0:DIM] = lo_ref[...].T
    dst_ref[:, DIM:2 * DIM] = hi_ref[...].T


def _linearize(table):
    t = table.T
    packed = pl.pallas_call(
        _tc_transpose_body,
        grid=(HBLK,),
        in_specs=[pl.BlockSpec((DIM, TBLK), lambda i: (0, i)),
                  pl.BlockSpec((DIM, TBLK), lambda i: (0, i + HBLK))],
        out_specs=pl.BlockSpec((TBLK, 2 * DIM), lambda i: (i, 0)),
        out_shape=jax.ShapeDtypeStruct((HALF, 2 * DIM), jnp.float32),
    )(t, t)
    return packed.reshape(2 * HALF, DIM)


def kernel(target_word_ids, context_word_ids, in_embedding, out_embedding):
    tgt = target_word_ids.reshape(BATCH).astype(jnp.int32)
    ctx = context_word_ids.reshape(BATCH * CTX).astype(jnp.int32)
    tgt = jnp.where(tgt < HALF, 2 * tgt, 2 * (tgt - HALF) + 1)
    ctx = jnp.where(ctx < HALF, 2 * ctx, 2 * (ctx - HALF) + 1)
    flat = _run(tgt, ctx, _linearize(in_embedding), _linearize(out_embedding))
    return flat.reshape(BATCH, CTX)
